# Initial kernel scaffold; baseline (speedup 1.0000x reference)
import jax
import jax.numpy as jnp
from jax.experimental import pallas as pl


def kernel(input_ids, table):
    raise NotImplementedError("write your pallas kernel here")


# speedup vs baseline: 1.3075x; 1.3075x over previous
import functools

import jax
import jax.numpy as jnp
from jax import lax
from jax.experimental import pallas as pl
from jax.experimental.pallas import tpu as pltpu
from jax.experimental.pallas import tpu_sc as plsc

VOCAB = 1000000
EMBED = 32
BATCH = 4096
SEQ = 200

NC, NS = 2, 16
NW = NC * NS
B = BATCH * SEQ
B_PER_W = B // NW
IG = 128
G_PER_W = B_PER_W // IG

_mesh = plsc.VectorSubcoreMesh(core_axis_name="c", subcore_axis_name="s")


@functools.partial(
    pl.kernel,
    out_type=jax.ShapeDtypeStruct((B, EMBED), jnp.float32),
    mesh=_mesh,
    scratch_types=[
        pltpu.VMEM((G_PER_W, IG), jnp.int32),
        pltpu.VMEM((IG, EMBED), jnp.float32),
        pltpu.SemaphoreType.DMA,
    ],
    compiler_params=pltpu.CompilerParams(use_tc_tiling_on_sc=False),
)
def _gather_kernel(idx_hbm, table_hbm, out_hbm, idx_v, rows_v, sem):
    wid = lax.axis_index("s") * NC + lax.axis_index("c")
    pltpu.sync_copy(idx_hbm.at[pl.ds(wid * G_PER_W, G_PER_W), :], idx_v)
    row_base = wid * B_PER_W

    def body(j, _):
        pltpu.async_copy(table_hbm.at[idx_v.at[j]], rows_v, sem).wait()
        pltpu.sync_copy(rows_v, out_hbm.at[pl.ds(row_base + j * IG, IG), :])
        return 0

    lax.fori_loop(0, G_PER_W, body, 0)


def kernel(input_ids, table):
    idx = input_ids.reshape(B // IG, IG).astype(jnp.int32)
    out = _gather_kernel(idx, table)
    return out.reshape(BATCH, SEQ, EMBED)


# speedup vs baseline: 1.4821x; 1.1335x over previous
import functools

import jax
import jax.numpy as jnp
from jax import lax
from jax.experimental import pallas as pl
from jax.experimental.pallas import tpu as pltpu
from jax.experimental.pallas import tpu_sc as plsc

VOCAB = 1000000
EMBED = 32
BATCH = 4096
SEQ = 200

NC, NS = 2, 16
NW = NC * NS
B = BATCH * SEQ
B_PER_W = B // NW
IG = 1280
G_PER_W = B_PER_W // IG

_mesh = plsc.VectorSubcoreMesh(core_axis_name="c", subcore_axis_name="s")


@functools.partial(
    pl.kernel,
    out_type=jax.ShapeDtypeStruct((B, EMBED), jnp.float32),
    mesh=_mesh,
    scratch_types=[
        pltpu.VMEM((G_PER_W, IG), jnp.int32),
        pltpu.VMEM((IG, EMBED), jnp.float32),
        pltpu.SemaphoreType.DMA,
    ],
    compiler_params=pltpu.CompilerParams(use_tc_tiling_on_sc=False),
)
def _gather_kernel(idx_hbm, table_hbm, out_hbm, idx_v, rows_v, sem):
    wid = lax.axis_index("s") * NC + lax.axis_index("c")
    pltpu.sync_copy(idx_hbm.at[pl.ds(wid * G_PER_W, G_PER_W), :], idx_v)
    row_base = wid * B_PER_W

    def body(j, _):
        pltpu.async_copy(table_hbm.at[idx_v.at[j]], rows_v, sem).wait()
        pltpu.sync_copy(rows_v, out_hbm.at[pl.ds(row_base + j * IG, IG), :])
        return 0

    lax.fori_loop(0, G_PER_W, body, 0)


def kernel(input_ids, table):
    idx = input_ids.reshape(B // IG, IG).astype(jnp.int32)
    out = _gather_kernel(idx, table)
    return out.reshape(BATCH, SEQ, EMBED)


# speedup vs baseline: 1.5019x; 1.0134x over previous
import functools

import jax
import jax.numpy as jnp
from jax import lax
from jax.experimental import pallas as pl
from jax.experimental.pallas import tpu as pltpu
from jax.experimental.pallas import tpu_sc as plsc

VOCAB = 1000000
EMBED = 32
BATCH = 4096
SEQ = 200

NC, NS = 2, 16
NW = NC * NS
B = BATCH * SEQ
B_PER_W = B // NW
IG = 1280
G_PER_W = B_PER_W // IG

_mesh = plsc.VectorSubcoreMesh(core_axis_name="c", subcore_axis_name="s")


@functools.partial(
    pl.kernel,
    out_type=jax.ShapeDtypeStruct((B, EMBED), jnp.float32),
    mesh=_mesh,
    scratch_types=[
        pltpu.VMEM((G_PER_W, IG), jnp.int32),
        pltpu.VMEM((IG, EMBED), jnp.float32),
        pltpu.VMEM((IG, EMBED), jnp.float32),
        pltpu.SemaphoreType.DMA,
        pltpu.SemaphoreType.DMA,
        pltpu.SemaphoreType.DMA,
        pltpu.SemaphoreType.DMA,
    ],
    compiler_params=pltpu.CompilerParams(use_tc_tiling_on_sc=False),
)
def _gather_kernel(idx_hbm, table_hbm, out_hbm, idx_v, rows_a, rows_b,
                   gsem_a, gsem_b, ssem_a, ssem_b):
    wid = lax.axis_index("s") * NC + lax.axis_index("c")
    pltpu.sync_copy(idx_hbm.at[pl.ds(wid * G_PER_W, G_PER_W), :], idx_v)
    row_base = wid * B_PER_W

    def gather(g, buf, sem):
        pltpu.async_copy(table_hbm.at[idx_v.at[g]], buf, sem)

    def wait_gather(g, buf, sem):
        pltpu.make_async_copy(table_hbm.at[idx_v.at[g]], buf, sem).wait()

    def store(g, buf, sem):
        pltpu.async_copy(buf, out_hbm.at[pl.ds(row_base + g * IG, IG), :], sem)

    def wait_store(g, buf, sem):
        pltpu.make_async_copy(
            buf, out_hbm.at[pl.ds(row_base + g * IG, IG), :], sem).wait()

    gather(0, rows_a, gsem_a)
    gather(1, rows_b, gsem_b)

    def body(t, _):
        g0 = 2 * t
        g1 = g0 + 1
        wait_gather(g0, rows_a, gsem_a)
        store(g0, rows_a, ssem_a)
        wait_store(g0, rows_a, ssem_a)
        gather(g0 + 2, rows_a, gsem_a)
        wait_gather(g1, rows_b, gsem_b)
        store(g1, rows_b, ssem_b)
        wait_store(g1, rows_b, ssem_b)
        gather(g1 + 2, rows_b, gsem_b)
        return 0

    lax.fori_loop(0, G_PER_W // 2 - 1, body, 0)

    g0 = G_PER_W - 2
    g1 = G_PER_W - 1
    wait_gather(g0, rows_a, gsem_a)
    store(g0, rows_a, ssem_a)
    wait_gather(g1, rows_b, gsem_b)
    store(g1, rows_b, ssem_b)
    wait_store(g0, rows_a, ssem_a)
    wait_store(g1, rows_b, ssem_b)


def kernel(input_ids, table):
    idx = input_ids.reshape(B // IG, IG).astype(jnp.int32)
    out = _gather_kernel(idx, table)
    return out.reshape(BATCH, SEQ, EMBED)
